# baseline (speedup 1.0000x reference)
import functools

import jax
import jax.numpy as jnp
from jax import lax
from jax.experimental import pallas as pl
from jax.experimental.pallas import tpu as pltpu
from jax.experimental.pallas import tpu_sc as plsc

_NC = 2
_NS = 16
_LANES = 16
_NW = _NC * _NS


def _rne_bf16_bits(mf):
    bits = lax.bitcast_convert_type(mf, jnp.int32)
    lsb = lax.shift_right_logical(bits, 16) & 1
    return lax.shift_right_logical(bits + 0x7FFF + lsb, 16)


def _matmul_stage(table, wt, v, d, lp):
    vh = v // 2
    rb = 10000

    def body(ta_ref, tb_ref, w_ref, q_ref):
        w = w_ref[...]
        pa = jnp.dot(ta_ref[...], w, preferred_element_type=jnp.float32)
        pb = jnp.dot(tb_ref[...], w, preferred_element_type=jnp.float32)
        packed = _rne_bf16_bits(pa) | (_rne_bf16_bits(pb) << 16)
        q_ref[...] = lax.bitcast_convert_type(packed, jnp.float32)

    return pl.pallas_call(
        body,
        grid=(vh // rb,),
        in_specs=[
            pl.BlockSpec((rb, d), lambda i: (i, 0)),
            pl.BlockSpec((rb, d), lambda i: (i + vh // rb, 0)),
            pl.BlockSpec((d, lp), lambda i: (0, 0)),
        ],
        out_specs=pl.BlockSpec((rb, lp), lambda i: (i, 0)),
        out_shape=jax.ShapeDtypeStruct((vh, lp), jnp.float32),
    )(table, table, wt)


def _transpose_stage(x, batch, seq):
    def body(x_ref, o_ref):
        o_ref[...] = x_ref[...].T

    return pl.pallas_call(
        body,
        out_shape=jax.ShapeDtypeStruct((seq, batch), jnp.int32),
    )(x)


def _gather_sum_stage(xt, q_flat, bvec, batch, seq, lp, vh):
    bpw = batch // _NW
    jg = bpw // _LANES

    mesh = plsc.VectorSubcoreMesh(core_axis_name="c", subcore_axis_name="s")

    @functools.partial(
        pl.kernel,
        out_type=jax.ShapeDtypeStruct((batch,), jnp.float32),
        mesh=mesh,
        scratch_types=[
            pltpu.VMEM((seq, bpw), jnp.int32),
            pltpu.VMEM((seq, bpw), jnp.int32),
            pltpu.VMEM((seq, bpw), jnp.float32),
            pltpu.VMEM((bpw,), jnp.float32),
            pltpu.VMEM((_LANES,), jnp.float32),
            pltpu.SemaphoreType.DMA,
        ],
    )
    def sc_kernel(xt_hbm, q_hbm, b_hbm, out_hbm, xv, idxv, gv, acc, bv, sem):
        wid = lax.axis_index("s") * _NC + lax.axis_index("c")
        base = wid * bpw
        pltpu.sync_copy(xt_hbm.at[:, pl.ds(base, bpw)], xv)
        pltpu.sync_copy(b_hbm, bv)

        def build(l, carry):
            for j in range(jg):
                sl = pl.ds(j * _LANES, _LANES)
                xw = xv[l, sl]
                sel = lax.shift_right_arithmetic(xw - vh, 31) + 1
                idxv[l, sl] = (xw - sel * vh) * lp + l
            return carry

        lax.fori_loop(0, seq, build, 0)

        def fire(l, carry):
            pltpu.async_copy(q_hbm.at[idxv.at[l]], gv.at[l], sem)
            return carry

        lax.fori_loop(0, seq, fire, 0)

        def drain(l, carry):
            pltpu.make_async_copy(q_hbm.at[idxv.at[l]], gv.at[l], sem).wait()
            return carry

        lax.fori_loop(0, seq, drain, 0)

        bias = bv[pl.ds(0, _LANES)]
        for j in range(jg):
            acc[pl.ds(j * _LANES, _LANES)] = bias

        def accum(l, carry):
            for j in range(jg):
                sl = pl.ds(j * _LANES, _LANES)
                sel = lax.shift_right_arithmetic(xv[l, sl] - vh, 31) + 1
                word = lax.bitcast_convert_type(gv[l, sl], jnp.int32)
                lo = (word << 16) & jnp.int32(-65536)
                hi = word & jnp.int32(-65536)
                fbits = lo + (hi - lo) * sel
                plsc.addupdate(acc.at[sl],
                               lax.bitcast_convert_type(fbits, jnp.float32))
            return carry

        lax.fori_loop(0, seq, accum, 0)

        pltpu.sync_copy(acc, out_hbm.at[pl.ds(base, bpw)])

    return sc_kernel(xt, q_flat, bvec)


def kernel(x, vocab_vectors, W, b):
    batch, seq = x.shape
    v, d = vocab_vectors.shape
    lp = 128
    vh = v // 2

    x = x.astype(jnp.int32)
    wt = W[:, 0].reshape(seq, d).T
    wt = jnp.pad(wt, ((0, 0), (0, lp - seq)))
    bvec = jnp.broadcast_to(b.astype(jnp.float32), (_LANES,))

    q = _matmul_stage(vocab_vectors, wt, v, d, lp)
    xt = _transpose_stage(x, batch, seq)
    out = _gather_sum_stage(xt, q.reshape(vh * lp), bvec, batch, seq, lp, vh)
    return out.reshape(batch, 1)


# speedup vs baseline: 1.1357x; 1.1357x over previous
import functools

import jax
import jax.numpy as jnp
from jax import lax
from jax.experimental import pallas as pl
from jax.experimental.pallas import tpu as pltpu
from jax.experimental.pallas import tpu_sc as plsc

_NC = 2
_NS = 16
_LANES = 16
_NW = _NC * _NS
_GRP = 10


def _rne_bf16_bits(mf):
    bits = lax.bitcast_convert_type(mf, jnp.int32)
    lsb = lax.shift_right_logical(bits, 16) & 1
    return lax.shift_right_logical(bits + 0x7FFF + lsb, 16)


def _matmul_stage(table, wt, v, d, lp):
    vh = v // 2
    rb = 10000

    def body(ta_ref, tb_ref, w_ref, q_ref):
        w = w_ref[...]
        pa = jnp.dot(ta_ref[...], w, preferred_element_type=jnp.float32)
        pb = jnp.dot(tb_ref[...], w, preferred_element_type=jnp.float32)
        packed = _rne_bf16_bits(pa) | (_rne_bf16_bits(pb) << 16)
        q_ref[...] = lax.bitcast_convert_type(packed, jnp.float32)

    return pl.pallas_call(
        body,
        grid=(vh // rb,),
        in_specs=[
            pl.BlockSpec((rb, d), lambda i: (i, 0)),
            pl.BlockSpec((rb, d), lambda i: (i + vh // rb, 0)),
            pl.BlockSpec((d, lp), lambda i: (0, 0)),
        ],
        out_specs=pl.BlockSpec((rb, lp), lambda i: (i, 0)),
        out_shape=jax.ShapeDtypeStruct((vh, lp), jnp.float32),
    )(table, table, wt)


def _gather_sum_stage(xt, q_flat, bvec, batch, seq, lp, vh):
    bpw = batch // _NW
    jg = bpw // _LANES

    mesh = plsc.VectorSubcoreMesh(core_axis_name="c", subcore_axis_name="s")

    @functools.partial(
        pl.kernel,
        out_type=jax.ShapeDtypeStruct((batch,), jnp.float32),
        mesh=mesh,
        scratch_types=[
            pltpu.VMEM((seq, bpw), jnp.int32),
            pltpu.VMEM((seq, bpw), jnp.int32),
            pltpu.VMEM((seq, bpw), jnp.float32),
            pltpu.VMEM((bpw,), jnp.float32),
            pltpu.VMEM((_LANES,), jnp.float32),
            pltpu.SemaphoreType.DMA((seq // _GRP,)),
        ],
    )
    def sc_kernel(xt_hbm, q_hbm, b_hbm, out_hbm, xv, idxv, gv, acc, bv, sems):
        wid = lax.axis_index("s") * _NC + lax.axis_index("c")
        base = wid * bpw
        pltpu.sync_copy(xt_hbm.at[:, pl.ds(base, bpw)], xv)
        pltpu.sync_copy(b_hbm, bv)

        def build(l, carry):
            for j in range(jg):
                sl = pl.ds(j * _LANES, _LANES)
                xw = xv[l, sl]
                sel = lax.shift_right_arithmetic(xw - vh, 31) + 1
                idxv[l, sl] = (xw - sel * vh) * lp + l
            return carry

        for g in range(seq // _GRP):
            lax.fori_loop(g * _GRP, (g + 1) * _GRP, build, 0)

            def fire(l, carry, _g=g):
                pltpu.async_copy(q_hbm.at[idxv.at[l]], gv.at[l], sems.at[_g])
                return carry

            lax.fori_loop(g * _GRP, (g + 1) * _GRP, fire, 0)

        bias = bv[pl.ds(0, _LANES)]
        for j in range(jg):
            acc[pl.ds(j * _LANES, _LANES)] = bias

        for g in range(seq // _GRP):

            def drain(l, carry, _g=g):
                pltpu.make_async_copy(q_hbm.at[idxv.at[l]], gv.at[l],
                                      sems.at[_g]).wait()
                return carry

            lax.fori_loop(g * _GRP, (g + 1) * _GRP, drain, 0)

            def accum(l, carry):
                for j in range(jg):
                    sl = pl.ds(j * _LANES, _LANES)
                    sel = lax.shift_right_arithmetic(xv[l, sl] - vh, 31) + 1
                    word = lax.bitcast_convert_type(gv[l, sl], jnp.int32)
                    lo = (word << 16) & jnp.int32(-65536)
                    hi = word & jnp.int32(-65536)
                    fbits = lo + (hi - lo) * sel
                    plsc.addupdate(acc.at[sl],
                                   lax.bitcast_convert_type(fbits, jnp.float32))
                return carry

            lax.fori_loop(g * _GRP, (g + 1) * _GRP, accum, 0)

        pltpu.sync_copy(acc, out_hbm.at[pl.ds(base, bpw)])

    return sc_kernel(xt, q_flat, bvec)


def kernel(x, vocab_vectors, W, b):
    batch, seq = x.shape
    v, d = vocab_vectors.shape
    lp = 128
    vh = v // 2

    x = x.astype(jnp.int32)
    wt = W[:, 0].reshape(seq, d).T
    wt = jnp.pad(wt, ((0, 0), (0, lp - seq)))
    bvec = jnp.broadcast_to(b.astype(jnp.float32), (_LANES,))

    q = _matmul_stage(vocab_vectors, wt, v, d, lp)
    out = _gather_sum_stage(x.T, q.reshape(vh * lp), bvec, batch, seq, lp, vh)
    return out.reshape(batch, 1)
